# baseline (speedup 1.0000x reference)
import functools

import jax
import jax.numpy as jnp
from jax import lax
from jax.experimental import pallas as pl
from jax.experimental.pallas import tpu as pltpu
from jax.experimental.pallas import tpu_sc as plsc

B = 65536
D = 129
L = 16
NC = 2
NS = 16
NW = NC * NS
ROWS_PER_W = B // NW
R = 128
NCHUNK = ROWS_PER_W // R
NPAIR = NCHUNK // 2
W = D - 1


def _body(in_hbm, out_hbm, in0, in1, out0, out1, si0, si1, so0, so1, scol):
    cid = lax.axis_index("c")
    sid = lax.axis_index("s")
    wid = sid * NC + cid
    base = wid * ROWS_PER_W

    pltpu.async_copy(
        in_hbm.at[pl.ds(base, ROWS_PER_W), pl.ds(W, 1)],
        out_hbm.at[pl.ds(base, ROWS_PER_W), pl.ds(W, 1)],
        scol,
    )

    in_bufs = (in0, in1)
    out_bufs = (out0, out1)
    isems = (si0, si1)
    osems = (so0, so1)

    iota = lax.iota(jnp.int32, L)
    srcs = [32 * k + 2 * iota for k in range(4)]
    srcs += [32 * k + 2 * iota + 1 for k in range(4)]

    pltpu.async_copy(in_hbm.at[pl.ds(base, R), pl.ds(0, W)], in0, si0)
    pltpu.async_copy(in_hbm.at[pl.ds(base + R, R), pl.ds(0, W)], in1, si1)

    def pair_body(t, carry):
        for b in range(2):
            c = 2 * t + b
            row0 = base + c * R
            iv, ov = in_bufs[b], out_bufs[b]
            isem, osem = isems[b], osems[b]

            pltpu.make_async_copy(
                in_hbm.at[pl.ds(row0, R), pl.ds(0, W)], iv, isem
            ).wait()

            @pl.when(t > 0)
            def _():
                pltpu.make_async_copy(
                    ov, out_hbm.at[pl.ds(row0, R), pl.ds(0, W)], osem
                ).wait()

            @plsc.parallel_loop(0, R, unroll=4)
            def row_body(r):
                rfull = jnp.full((L,), r, jnp.int32)
                for k in range(8):
                    ov[r, pl.ds(k * L, L)] = plsc.load_gather(
                        iv, [rfull, srcs[k]]
                    )

            @pl.when(t < NPAIR - 1)
            def _():
                pltpu.async_copy(
                    in_hbm.at[pl.ds(row0 + 2 * R, R), pl.ds(0, W)], iv, isem
                )

            pltpu.async_copy(ov, out_hbm.at[pl.ds(row0, R), pl.ds(0, W)], osem)
        return carry

    lax.fori_loop(0, NPAIR, pair_body, 0)

    pltpu.make_async_copy(
        out0, out_hbm.at[pl.ds(base, R), pl.ds(0, W)], so0
    ).wait()
    pltpu.make_async_copy(
        out1, out_hbm.at[pl.ds(base + R, R), pl.ds(0, W)], so1
    ).wait()
    pltpu.make_async_copy(
        in_hbm.at[pl.ds(base, ROWS_PER_W), pl.ds(W, 1)],
        out_hbm.at[pl.ds(base, ROWS_PER_W), pl.ds(W, 1)],
        scol,
    ).wait()


@jax.jit
def kernel(tensor):
    mesh = plsc.VectorSubcoreMesh(core_axis_name="c", subcore_axis_name="s")
    f = functools.partial(
        pl.kernel,
        mesh=mesh,
        out_type=jax.ShapeDtypeStruct((B, D), jnp.float32),
        scratch_types=[
            pltpu.VMEM((R, W), jnp.float32),
            pltpu.VMEM((R, W), jnp.float32),
            pltpu.VMEM((R, W), jnp.float32),
            pltpu.VMEM((R, W), jnp.float32),
            pltpu.SemaphoreType.DMA,
            pltpu.SemaphoreType.DMA,
            pltpu.SemaphoreType.DMA,
            pltpu.SemaphoreType.DMA,
            pltpu.SemaphoreType.DMA,
        ],
        compiler_params=pltpu.CompilerParams(
            use_tc_tiling_on_sc=True, needs_layout_passes=False
        ),
    )(_body)
    return f(tensor)


# speedup vs baseline: 7.1148x; 7.1148x over previous
import functools

import jax
import jax.numpy as jnp
from jax import lax
from jax.experimental import pallas as pl
from jax.experimental.pallas import tpu as pltpu
from jax.experimental.pallas import tpu_sc as plsc

B = 65536
D = 129
L = 16
NC = 2
NS = 16
NW = NC * NS
ROWS_PER_W = B // NW
R = 128
NCHUNK = ROWS_PER_W // R
NPAIR = NCHUNK // 2


def _body(in_hbm, out_hbm, in0, in1, out0, out1, si0, si1, so0, so1):
    cid = lax.axis_index("c")
    sid = lax.axis_index("s")
    wid = sid * NC + cid
    base = wid * ROWS_PER_W

    in_bufs = (in0, in1)
    out_bufs = (out0, out1)
    isems = (si0, si1)
    osems = (so0, so1)

    iota = lax.iota(jnp.int32, L)
    srcs = [32 * k + 2 * iota for k in range(4)]
    srcs += [32 * k + 2 * iota + 1 for k in range(4)]
    col_last = jnp.full((L,), D - 1, jnp.int32)

    pltpu.async_copy(in_hbm.at[pl.ds(base, R)], in0, si0)
    pltpu.async_copy(in_hbm.at[pl.ds(base + R, R)], in1, si1)

    def pair_body(t, carry):
        for b in range(2):
            c = 2 * t + b
            row0 = base + c * R
            iv, ov = in_bufs[b], out_bufs[b]
            isem, osem = isems[b], osems[b]

            pltpu.make_async_copy(in_hbm.at[pl.ds(row0, R)], iv, isem).wait()

            @pl.when(t > 0)
            def _():
                pltpu.make_async_copy(
                    ov, out_hbm.at[pl.ds(row0, R)], osem
                ).wait()

            @plsc.parallel_loop(0, R, unroll=4)
            def row_body(r):
                rfull = jnp.full((L,), r, jnp.int32)
                for k in range(8):
                    ov[r, pl.ds(k * L, L)] = plsc.load_gather(
                        iv, [rfull, srcs[k]]
                    )

            @plsc.parallel_loop(0, R, step=L, unroll=2)
            def tail_body(tt):
                rows = tt + iota
                val = plsc.load_gather(iv, [rows, col_last])
                plsc.store_scatter(ov, [rows, col_last], val)

            @pl.when(t < NPAIR - 1)
            def _():
                pltpu.async_copy(
                    in_hbm.at[pl.ds(row0 + 2 * R, R)], iv, isem
                )

            pltpu.async_copy(ov, out_hbm.at[pl.ds(row0, R)], osem)
        return carry

    lax.fori_loop(0, NPAIR, pair_body, 0)

    pltpu.make_async_copy(out0, out_hbm.at[pl.ds(base, R)], so0).wait()
    pltpu.make_async_copy(out1, out_hbm.at[pl.ds(base + R, R)], so1).wait()


@jax.jit
def kernel(tensor):
    mesh = plsc.VectorSubcoreMesh(core_axis_name="c", subcore_axis_name="s")
    f = functools.partial(
        pl.kernel,
        mesh=mesh,
        out_type=jax.ShapeDtypeStruct((B, D), jnp.float32),
        scratch_types=[
            pltpu.VMEM((R, D), jnp.float32),
            pltpu.VMEM((R, D), jnp.float32),
            pltpu.VMEM((R, D), jnp.float32),
            pltpu.VMEM((R, D), jnp.float32),
            pltpu.SemaphoreType.DMA,
            pltpu.SemaphoreType.DMA,
            pltpu.SemaphoreType.DMA,
            pltpu.SemaphoreType.DMA,
        ],
        compiler_params=pltpu.CompilerParams(
            use_tc_tiling_on_sc=True, needs_layout_passes=False
        ),
    )(_body)
    return f(tensor)


# speedup vs baseline: 7.1348x; 1.0028x over previous
import functools

import jax
import jax.numpy as jnp
from jax import lax
from jax.experimental import pallas as pl
from jax.experimental.pallas import tpu as pltpu
from jax.experimental.pallas import tpu_sc as plsc

B = 65536
D = 129
L = 16
NC = 2
NS = 16
NW = NC * NS
ROWS_PER_W = B // NW
R = 128
NCHUNK = ROWS_PER_W // R
NPAIR = NCHUNK // 2
W = D - 1


def _body(in_hbm, out_hbm, *scr):
    (in0, in1, out0, out1, ic0, ic1,
     si0, si1, so0, so1, sci0, sci1, sco0, sco1) = scr

    cid = lax.axis_index("c")
    sid = lax.axis_index("s")
    wid = sid * NC + cid
    base = wid * ROWS_PER_W

    in_bufs = (in0, in1)
    out_bufs = (out0, out1)
    icols = (ic0, ic1)
    isems = (si0, si1)
    osems = (so0, so1)
    cisems = (sci0, sci1)
    cosems = (sco0, sco1)

    iota = lax.iota(jnp.int32, L)
    srcs = [32 * k + 2 * iota for k in range(4)]
    srcs += [32 * k + 2 * iota + 1 for k in range(4)]

    for b in range(2):
        pltpu.async_copy(
            in_hbm.at[pl.ds(base + b * R, R), pl.ds(0, W)], in_bufs[b], isems[b]
        )
        pltpu.async_copy(
            in_hbm.at[pl.ds(base + b * R, R), pl.ds(W, 1)], icols[b], cisems[b]
        )

    def pair_body(t, carry):
        for b in range(2):
            c = 2 * t + b
            row0 = base + c * R
            iv, ov = in_bufs[b], out_bufs[b]
            isem, osem = isems[b], osems[b]

            pltpu.make_async_copy(
                in_hbm.at[pl.ds(row0, R), pl.ds(W, 1)], icols[b], cisems[b]
            ).wait()
            pltpu.async_copy(
                icols[b], out_hbm.at[pl.ds(row0, R), pl.ds(W, 1)], cosems[b]
            )

            pltpu.make_async_copy(
                in_hbm.at[pl.ds(row0, R), pl.ds(0, W)], iv, isem
            ).wait()

            @pl.when(t > 0)
            def _():
                pltpu.make_async_copy(
                    ov, out_hbm.at[pl.ds(row0, R), pl.ds(0, W)], osem
                ).wait()

            @plsc.parallel_loop(0, R, unroll=4)
            def row_body(r):
                rfull = jnp.full((L,), r, jnp.int32)
                for k in range(8):
                    ov[r, pl.ds(k * L, L)] = plsc.load_gather(
                        iv, [rfull, srcs[k]]
                    )

            @pl.when(t < NPAIR - 1)
            def _():
                pltpu.async_copy(
                    in_hbm.at[pl.ds(row0 + 2 * R, R), pl.ds(0, W)], iv, isem
                )

            pltpu.async_copy(ov, out_hbm.at[pl.ds(row0, R), pl.ds(0, W)], osem)

            pltpu.make_async_copy(
                icols[b], out_hbm.at[pl.ds(row0, R), pl.ds(W, 1)], cosems[b]
            ).wait()

            @pl.when(t < NPAIR - 1)
            def _():
                pltpu.async_copy(
                    in_hbm.at[pl.ds(row0 + 2 * R, R), pl.ds(W, 1)],
                    icols[b],
                    cisems[b],
                )
        return carry

    lax.fori_loop(0, NPAIR, pair_body, 0)

    for b in range(2):
        row0 = base + (NCHUNK - 2 + b) * R
        pltpu.make_async_copy(
            out_bufs[b], out_hbm.at[pl.ds(row0, R), pl.ds(0, W)], osems[b]
        ).wait()


@jax.jit
def kernel(tensor):
    mesh = plsc.VectorSubcoreMesh(core_axis_name="c", subcore_axis_name="s")
    f = functools.partial(
        pl.kernel,
        mesh=mesh,
        out_type=jax.ShapeDtypeStruct((B, D), jnp.float32),
        scratch_types=[
            pltpu.VMEM((R, W), jnp.float32),
            pltpu.VMEM((R, W), jnp.float32),
            pltpu.VMEM((R, W), jnp.float32),
            pltpu.VMEM((R, W), jnp.float32),
            pltpu.VMEM((R, 1), jnp.float32),
            pltpu.VMEM((R, 1), jnp.float32),
            pltpu.SemaphoreType.DMA,
            pltpu.SemaphoreType.DMA,
            pltpu.SemaphoreType.DMA,
            pltpu.SemaphoreType.DMA,
            pltpu.SemaphoreType.DMA,
            pltpu.SemaphoreType.DMA,
            pltpu.SemaphoreType.DMA,
            pltpu.SemaphoreType.DMA,
        ],
        compiler_params=pltpu.CompilerParams(
            use_tc_tiling_on_sc=True, needs_layout_passes=False
        ),
    )(_body)
    return f(tensor)
